# baseline (speedup 1.0000x reference)
import functools

import jax
import jax.numpy as jnp
from jax import lax
from jax.experimental import pallas as pl
from jax.experimental.pallas import tpu as pltpu
from jax.experimental.pallas import tpu_sc as plsc

_DN_T = (((1,), (1,)), ((), ()))

_NEG = -3.0e38


def _gate_body(x_ref, Wg_ref, bg_ref, glT_ref):
    glT_ref[...] = lax.dot_general(
        Wg_ref[...], x_ref[...], _DN_T, preferred_element_type=jnp.float32
    ) + bg_ref[...][:, None]


def _route_body(E, TPW, glT_hbm, fullT_hbm, sparseT_hbm, gl_v, full_v, sparse_v):
    wid = lax.axis_index("s") * 2 + lax.axis_index("c")
    base = wid * TPW
    pltpu.sync_copy(glT_hbm.at[:, pl.ds(base, TPW)], gl_v)
    for c in range(TPW // 16):
        sl = pl.ds(c * 16, 16)
        p = [gl_v[e, sl] for e in range(E)]
        m = p[0]
        for e in range(1, E):
            m = jnp.maximum(m, p[e])
        eg = [jnp.exp(p[e] - m) for e in range(E)]
        ssum = eg[0]
        for e in range(1, E):
            ssum = ssum + eg[e]
        fp = [eg[e] / ssum for e in range(E)]
        for e in range(E):
            full_v[e, sl] = fp[e]
        v1 = p[0]
        for e in range(1, E):
            v1 = jnp.maximum(v1, p[e])
        i1 = jnp.full((16,), E, jnp.int32)
        for e in range(E - 1, -1, -1):
            i1 = jnp.where(p[e] == v1, e, i1)
        p2 = [jnp.where(i1 == e, _NEG, p[e]) for e in range(E)]
        v2 = p2[0]
        for e in range(1, E):
            v2 = jnp.maximum(v2, p2[e])
        i2 = jnp.full((16,), E, jnp.int32)
        for e in range(E - 1, -1, -1):
            i2 = jnp.where(p2[e] == v2, e, i2)
        pv1 = fp[0]
        pv2 = jnp.where(i1 == 0, _NEG, fp[0])
        for e in range(1, E):
            pv1 = jnp.maximum(pv1, fp[e])
            pv2 = jnp.maximum(pv2, jnp.where(i1 == e, _NEG, fp[e]))
        s = pv1 + pv2
        w1 = pv1 / s
        w2 = pv2 / s
        for e in range(E):
            sparse_v[e, sl] = jnp.where(
                i1 == e, w1, jnp.where(i2 == e, w2, 0.0))
    pltpu.sync_copy(full_v, fullT_hbm.at[:, pl.ds(base, TPW)])
    pltpu.sync_copy(sparse_v, sparseT_hbm.at[:, pl.ds(base, TPW)])


def _moe_body(H, x_ref, Wef_ref, bef_ref, Wc_ref, bc_ref, spT_ref, fullT_ref,
              logits_ref, sparse_ref, mixed_ref, full_ref):
    E = spT_ref.shape[0]
    xt = x_ref[...]
    sp = spT_ref[...].T
    sparse_ref[...] = sp
    full_ref[...] = fullT_ref[...].T

    z_all = lax.dot_general(xt, Wef_ref[...], _DN_T,
                            preferred_element_type=jnp.float32)
    h_all = jnp.maximum(z_all + bef_ref[...][None, :], 0.0)
    acc = sp[:, 0:1] * h_all[:, 0:H]
    for e in range(1, E):
        acc = acc + sp[:, e:e + 1] * h_all[:, e * H:(e + 1) * H]
    mixed_ref[...] = acc

    logits_ref[...] = (
        lax.dot_general(acc, Wc_ref[...], _DN_T,
                        preferred_element_type=jnp.float32)
        + bc_ref[...][None, :]
    )


def kernel(x, Wg, bg, We, be, Wc, bc):
    B, D = x.shape
    E, H, _ = We.shape
    C = Wc.shape[0]
    Wef = We.reshape(E * H, D)
    bef = be.reshape(E * H)

    BT = 512 if B % 512 == 0 else B
    grid = (B // BT,)

    glT = pl.pallas_call(
        _gate_body,
        grid=grid,
        in_specs=[
            pl.BlockSpec((BT, D), lambda i: (i, 0)),
            pl.BlockSpec((E, D), lambda i: (0, 0)),
            pl.BlockSpec((E,), lambda i: (0,)),
        ],
        out_specs=pl.BlockSpec((E, BT), lambda i: (0, i)),
        out_shape=jax.ShapeDtypeStruct((E, B), jnp.float32),
    )(x, Wg, bg)

    info = plsc.get_sparse_core_info()
    NW = info.num_cores * info.num_subcores
    TPW = B // NW
    mesh = plsc.VectorSubcoreMesh(core_axis_name="c", subcore_axis_name="s")
    fullT, sparseT = pl.kernel(
        functools.partial(_route_body, E, TPW),
        mesh=mesh,
        out_type=[
            jax.ShapeDtypeStruct((E, B), jnp.float32),
            jax.ShapeDtypeStruct((E, B), jnp.float32),
        ],
        scratch_types=[
            pltpu.VMEM((E, TPW), jnp.float32),
            pltpu.VMEM((E, TPW), jnp.float32),
            pltpu.VMEM((E, TPW), jnp.float32),
        ],
    )(glT)

    logits, sparse, mixed, full = pl.pallas_call(
        functools.partial(_moe_body, H),
        grid=grid,
        in_specs=[
            pl.BlockSpec((BT, D), lambda i: (i, 0)),
            pl.BlockSpec((E * H, D), lambda i: (0, 0)),
            pl.BlockSpec((E * H,), lambda i: (0,)),
            pl.BlockSpec((C, H), lambda i: (0, 0)),
            pl.BlockSpec((C,), lambda i: (0,)),
            pl.BlockSpec((E, BT), lambda i: (0, i)),
            pl.BlockSpec((E, BT), lambda i: (0, i)),
        ],
        out_specs=[
            pl.BlockSpec((BT, C), lambda i: (i, 0)),
            pl.BlockSpec((BT, E), lambda i: (i, 0)),
            pl.BlockSpec((BT, H), lambda i: (i, 0)),
            pl.BlockSpec((BT, E), lambda i: (i, 0)),
        ],
        out_shape=[
            jax.ShapeDtypeStruct((B, C), jnp.float32),
            jax.ShapeDtypeStruct((B, E), jnp.float32),
            jax.ShapeDtypeStruct((B, H), jnp.float32),
            jax.ShapeDtypeStruct((B, E), jnp.float32),
        ],
    )(x, Wef, bef, Wc, bc, sparseT, fullT)

    return (logits, sparse, mixed, full)


# speedup vs baseline: 1.5257x; 1.5257x over previous
import jax
import jax.numpy as jnp
from jax import lax
from jax.experimental import pallas as pl

_DN_T = (((1,), (1,)), ((), ()))


def _moe_body(x_ref, Wg_ref, bg_ref, We_ref, be_ref, Wc_ref, bc_ref,
              logitsT_ref, sparseT_ref, mixed_ref, fullT_ref):
    E = We_ref.shape[0]
    BT = x_ref.shape[0]
    xt = x_ref[...]

    gl = lax.dot_general(xt, Wg_ref[...], _DN_T,
                         preferred_element_type=jnp.float32)
    gl = gl + bg_ref[...][None, :]
    m = jnp.max(gl, axis=1, keepdims=True)
    eg = jnp.exp(gl - m)
    probs = eg / jnp.sum(eg, axis=1, keepdims=True)
    fullT_ref[...] = probs.T

    e_iota = lax.broadcasted_iota(jnp.int32, (BT, E), 1)
    v1 = jnp.max(probs, axis=1, keepdims=True)
    i1 = jnp.min(jnp.where(probs == v1, e_iota, E), axis=1, keepdims=True)
    probs2 = jnp.where(e_iota == i1, -1.0, probs)
    v2 = jnp.max(probs2, axis=1, keepdims=True)
    i2 = jnp.min(jnp.where(probs2 == v2, e_iota, E), axis=1, keepdims=True)
    s = v1 + v2
    sparse = jnp.where(e_iota == i1, v1 / s, 0.0) + jnp.where(e_iota == i2, v2 / s, 0.0)
    sparseT_ref[...] = sparse.T

    acc = jnp.zeros((BT, We_ref.shape[1]), jnp.float32)
    for e in range(E):
        h = lax.dot_general(xt, We_ref[e], _DN_T,
                            preferred_element_type=jnp.float32)
        h = jnp.maximum(h + be_ref[e][None, :], 0.0)
        acc = acc + sparse[:, e:e + 1] * h
    mixed_ref[...] = acc

    logitsT_ref[...] = lax.dot_general(
        Wc_ref[...], acc, _DN_T, preferred_element_type=jnp.float32
    ) + bc_ref[...][:, None]


def kernel(x, Wg, bg, We, be, Wc, bc):
    B, D = x.shape
    E, H, _ = We.shape
    C = Wc.shape[0]

    BT = 512 if B % 512 == 0 else B
    grid = (B // BT,)

    logitsT, sparseT, mixed, fullT = pl.pallas_call(
        _moe_body,
        grid=grid,
        in_specs=[
            pl.BlockSpec((BT, D), lambda i: (i, 0)),
            pl.BlockSpec((E, D), lambda i: (0, 0)),
            pl.BlockSpec((E,), lambda i: (0,)),
            pl.BlockSpec((E, H, D), lambda i: (0, 0, 0)),
            pl.BlockSpec((E, H), lambda i: (0, 0)),
            pl.BlockSpec((C, H), lambda i: (0, 0)),
            pl.BlockSpec((C,), lambda i: (0,)),
        ],
        out_specs=[
            pl.BlockSpec((C, BT), lambda i: (0, i)),
            pl.BlockSpec((E, BT), lambda i: (0, i)),
            pl.BlockSpec((BT, H), lambda i: (i, 0)),
            pl.BlockSpec((E, BT), lambda i: (0, i)),
        ],
        out_shape=[
            jax.ShapeDtypeStruct((C, B), jnp.float32),
            jax.ShapeDtypeStruct((E, B), jnp.float32),
            jax.ShapeDtypeStruct((B, H), jnp.float32),
            jax.ShapeDtypeStruct((E, B), jnp.float32),
        ],
    )(x, Wg, bg, We, be, Wc, bc)

    return (logitsT.T, sparseT.T, mixed, fullT.T)


# speedup vs baseline: 1.5811x; 1.0363x over previous
import jax
import jax.numpy as jnp
from jax import lax
from jax.experimental import pallas as pl

_DN_T = (((1,), (1,)), ((), ()))


def _moe_body(x_ref, Wg_ref, bg_ref, We_ref, be_ref, Wc_ref, bc_ref,
              logitsT_ref, sparseT_ref, mixed_ref, fullT_ref):
    E = We_ref.shape[0]
    BT = x_ref.shape[0]
    xt = x_ref[...]

    gl = lax.dot_general(xt, Wg_ref[...], _DN_T,
                         preferred_element_type=jnp.float32)
    gl = gl + bg_ref[...][None, :]
    m = jnp.max(gl, axis=1, keepdims=True)
    eg = jnp.exp(gl - m)
    probs = eg / jnp.sum(eg, axis=1, keepdims=True)
    fullT_ref[...] = probs.T

    e_iota = lax.broadcasted_iota(jnp.int32, (BT, E), 1)
    v1 = jnp.max(probs, axis=1, keepdims=True)
    i1 = jnp.min(jnp.where(probs == v1, e_iota, E), axis=1, keepdims=True)
    probs2 = jnp.where(e_iota == i1, -1.0, probs)
    v2 = jnp.max(probs2, axis=1, keepdims=True)
    i2 = jnp.min(jnp.where(probs2 == v2, e_iota, E), axis=1, keepdims=True)
    s = v1 + v2
    sparse = jnp.where(e_iota == i1, v1 / s, 0.0) + jnp.where(e_iota == i2, v2 / s, 0.0)
    sparseT_ref[...] = sparse.T

    acc = jnp.zeros((BT, We_ref.shape[1]), jnp.float32)
    for e in range(E):
        h = lax.dot_general(xt, We_ref[e], _DN_T,
                            preferred_element_type=jnp.float32)
        h = jnp.maximum(h + be_ref[e][None, :], 0.0)
        acc = acc + sparse[:, e:e + 1] * h
    mixed_ref[...] = acc

    logitsT_ref[...] = lax.dot_general(
        Wc_ref[...], acc, _DN_T, preferred_element_type=jnp.float32
    ) + bc_ref[...][:, None]


def kernel(x, Wg, bg, We, be, Wc, bc):
    B, D = x.shape
    E, H, _ = We.shape
    C = Wc.shape[0]

    BT = 1024 if B % 1024 == 0 else B
    grid = (B // BT,)

    logitsT, sparseT, mixed, fullT = pl.pallas_call(
        _moe_body,
        grid=grid,
        in_specs=[
            pl.BlockSpec((BT, D), lambda i: (i, 0)),
            pl.BlockSpec((E, D), lambda i: (0, 0)),
            pl.BlockSpec((E,), lambda i: (0,)),
            pl.BlockSpec((E, H, D), lambda i: (0, 0, 0)),
            pl.BlockSpec((E, H), lambda i: (0, 0)),
            pl.BlockSpec((C, H), lambda i: (0, 0)),
            pl.BlockSpec((C,), lambda i: (0,)),
        ],
        out_specs=[
            pl.BlockSpec((C, BT), lambda i: (0, i)),
            pl.BlockSpec((E, BT), lambda i: (0, i)),
            pl.BlockSpec((BT, H), lambda i: (i, 0)),
            pl.BlockSpec((E, BT), lambda i: (0, i)),
        ],
        out_shape=[
            jax.ShapeDtypeStruct((C, B), jnp.float32),
            jax.ShapeDtypeStruct((E, B), jnp.float32),
            jax.ShapeDtypeStruct((B, H), jnp.float32),
            jax.ShapeDtypeStruct((E, B), jnp.float32),
        ],
    )(x, Wg, bg, We, be, Wc, bc)

    return (logitsT.T, sparseT.T, mixed, fullT.T)


# speedup vs baseline: 1.5821x; 1.0006x over previous
import jax
import jax.numpy as jnp
from jax import lax
from jax.experimental import pallas as pl
from jax.experimental.pallas import tpu as pltpu

_DN_T = (((1,), (1,)), ((), ()))


def _moe_body(x_ref, Wg_ref, bg_ref, We_ref, be_ref, Wc_ref, bc_ref,
              logitsT_ref, sparseT_ref, mixed_ref, fullT_ref):
    E = We_ref.shape[0]
    BT = x_ref.shape[0]
    xt = x_ref[...]

    gl = lax.dot_general(xt, Wg_ref[...], _DN_T,
                         preferred_element_type=jnp.float32)
    gl = gl + bg_ref[...][None, :]
    m = jnp.max(gl, axis=1, keepdims=True)
    eg = jnp.exp(gl - m)
    probs = eg / jnp.sum(eg, axis=1, keepdims=True)
    fullT_ref[...] = probs.T

    e_iota = lax.broadcasted_iota(jnp.int32, (BT, E), 1)
    v1 = jnp.max(probs, axis=1, keepdims=True)
    i1 = jnp.min(jnp.where(probs == v1, e_iota, E), axis=1, keepdims=True)
    probs2 = jnp.where(e_iota == i1, -1.0, probs)
    v2 = jnp.max(probs2, axis=1, keepdims=True)
    i2 = jnp.min(jnp.where(probs2 == v2, e_iota, E), axis=1, keepdims=True)
    s = v1 + v2
    sparse = jnp.where(e_iota == i1, v1 / s, 0.0) + jnp.where(e_iota == i2, v2 / s, 0.0)
    sparseT_ref[...] = sparse.T

    acc = jnp.zeros((BT, We_ref.shape[1]), jnp.float32)
    for e in range(E):
        h = lax.dot_general(xt, We_ref[e], _DN_T,
                            preferred_element_type=jnp.float32)
        h = jnp.maximum(h + be_ref[e][None, :], 0.0)
        acc = acc + sparse[:, e:e + 1] * h
    mixed_ref[...] = acc

    logitsT_ref[...] = lax.dot_general(
        Wc_ref[...], acc, _DN_T, preferred_element_type=jnp.float32
    ) + bc_ref[...][:, None]


def kernel(x, Wg, bg, We, be, Wc, bc):
    B, D = x.shape
    E, H, _ = We.shape
    C = Wc.shape[0]

    BT = 1024 if B % 1024 == 0 else B
    grid = (B // BT,)

    logitsT, sparseT, mixed, fullT = pl.pallas_call(
        _moe_body,
        grid=grid,
        compiler_params=pltpu.CompilerParams(
            dimension_semantics=("parallel",)),
        in_specs=[
            pl.BlockSpec((BT, D), lambda i: (i, 0)),
            pl.BlockSpec((E, D), lambda i: (0, 0)),
            pl.BlockSpec((E,), lambda i: (0,)),
            pl.BlockSpec((E, H, D), lambda i: (0, 0, 0)),
            pl.BlockSpec((E, H), lambda i: (0, 0)),
            pl.BlockSpec((C, H), lambda i: (0, 0)),
            pl.BlockSpec((C,), lambda i: (0,)),
        ],
        out_specs=[
            pl.BlockSpec((C, BT), lambda i: (0, i)),
            pl.BlockSpec((E, BT), lambda i: (0, i)),
            pl.BlockSpec((BT, H), lambda i: (i, 0)),
            pl.BlockSpec((E, BT), lambda i: (0, i)),
        ],
        out_shape=[
            jax.ShapeDtypeStruct((C, B), jnp.float32),
            jax.ShapeDtypeStruct((E, B), jnp.float32),
            jax.ShapeDtypeStruct((B, H), jnp.float32),
            jax.ShapeDtypeStruct((E, B), jnp.float32),
        ],
    )(x, Wg, bg, We, be, Wc, bc)

    return (logitsT.T, sparseT.T, mixed, fullT.T)


# speedup vs baseline: 1.6174x; 1.0223x over previous
import jax
import jax.numpy as jnp
from jax import lax
from jax.experimental import pallas as pl
from jax.experimental.pallas import tpu as pltpu

_DN_T = (((1,), (1,)), ((), ()))


def _moe_body(x_ref, Wg_ref, bg_ref, We_ref, be_ref, Wc_ref, bc_ref,
              logitsT_ref, sparseT_ref, mixed_ref, fullT_ref):
    E = We_ref.shape[0]
    BT = x_ref.shape[0]
    xt = x_ref[...]

    gl = lax.dot_general(xt, Wg_ref[...], _DN_T,
                         preferred_element_type=jnp.float32)
    gl = gl + bg_ref[...][None, :]
    m = jnp.max(gl, axis=1, keepdims=True)
    eg = jnp.exp(gl - m)
    probs = eg / jnp.sum(eg, axis=1, keepdims=True)
    fullT_ref[...] = probs.T

    e_iota = lax.broadcasted_iota(jnp.int32, (BT, E), 1)
    v1 = jnp.max(probs, axis=1, keepdims=True)
    i1 = jnp.min(jnp.where(probs == v1, e_iota, E), axis=1, keepdims=True)
    probs2 = jnp.where(e_iota == i1, -1.0, probs)
    v2 = jnp.max(probs2, axis=1, keepdims=True)
    i2 = jnp.min(jnp.where(probs2 == v2, e_iota, E), axis=1, keepdims=True)
    s = v1 + v2
    sparse = jnp.where(e_iota == i1, v1 / s, 0.0) + jnp.where(e_iota == i2, v2 / s, 0.0)
    sparseT_ref[...] = sparse.T

    H = We_ref.shape[1]
    acc_a = jnp.zeros((BT, H), jnp.float32)
    acc_b = jnp.zeros((BT, H), jnp.float32)
    for e in range(0, E, 2):
        ha = lax.dot_general(xt, We_ref[e], _DN_T,
                             preferred_element_type=jnp.float32)
        hb = lax.dot_general(xt, We_ref[e + 1], _DN_T,
                             preferred_element_type=jnp.float32)
        ha = jnp.maximum(ha + be_ref[e][None, :], 0.0)
        hb = jnp.maximum(hb + be_ref[e + 1][None, :], 0.0)
        acc_a = acc_a + sparse[:, e:e + 1] * ha
        acc_b = acc_b + sparse[:, e + 1:e + 2] * hb
    acc = acc_a + acc_b
    mixed_ref[...] = acc

    logitsT_ref[...] = lax.dot_general(
        Wc_ref[...], acc, _DN_T, preferred_element_type=jnp.float32
    ) + bc_ref[...][:, None]


def kernel(x, Wg, bg, We, be, Wc, bc):
    B, D = x.shape
    E, H, _ = We.shape
    C = Wc.shape[0]

    BT = 1024 if B % 1024 == 0 else B
    grid = (B // BT,)

    logitsT, sparseT, mixed, fullT = pl.pallas_call(
        _moe_body,
        grid=grid,
        compiler_params=pltpu.CompilerParams(
            dimension_semantics=("parallel",)),
        in_specs=[
            pl.BlockSpec((BT, D), lambda i: (i, 0)),
            pl.BlockSpec((E, D), lambda i: (0, 0)),
            pl.BlockSpec((E,), lambda i: (0,)),
            pl.BlockSpec((E, H, D), lambda i: (0, 0, 0)),
            pl.BlockSpec((E, H), lambda i: (0, 0)),
            pl.BlockSpec((C, H), lambda i: (0, 0)),
            pl.BlockSpec((C,), lambda i: (0,)),
        ],
        out_specs=[
            pl.BlockSpec((C, BT), lambda i: (0, i)),
            pl.BlockSpec((E, BT), lambda i: (0, i)),
            pl.BlockSpec((BT, H), lambda i: (i, 0)),
            pl.BlockSpec((E, BT), lambda i: (0, i)),
        ],
        out_shape=[
            jax.ShapeDtypeStruct((C, B), jnp.float32),
            jax.ShapeDtypeStruct((E, B), jnp.float32),
            jax.ShapeDtypeStruct((B, H), jnp.float32),
            jax.ShapeDtypeStruct((E, B), jnp.float32),
        ],
    )(x, Wg, bg, We, be, Wc, bc)

    return (logitsT.T, sparseT.T, mixed, fullT.T)
